# baseline (speedup 1.0000x reference)
import functools

import jax
import jax.numpy as jnp
from jax import lax
from jax.experimental import pallas as pl
from jax.experimental.pallas import tpu as pltpu
from jax.experimental.pallas import tpu_sc as plsc

B, L = 1024, 200
T = B * L
HID = 128
INTD = 42
HALF = 64

NC, NS = 2, 16
NW = NC * NS
TPW = T // NW
CHUNK = 128
NCHUNK = TPW // CHUNK

LN_EPS = 1e-6


def _project_body(test_ref, q_ref, tag_ref, tt_ref, w_ref, b_ref,
                  ot_ref, oq_ref, og_ref, ott_ref):
    w = w_ref[...]
    dn = (((1,), (1,)), ((), ()))
    ot_ref[...] = lax.dot_general(test_ref[...], w[:, 0 * INTD:1 * INTD], dn,
                                  preferred_element_type=jnp.float32
                                  ).astype(jnp.bfloat16)
    oq_ref[...] = lax.dot_general(q_ref[...], w[:, 1 * INTD:2 * INTD], dn,
                                  preferred_element_type=jnp.float32
                                  ).astype(jnp.bfloat16)
    og_ref[...] = lax.dot_general(tag_ref[...], w[:, 2 * INTD:3 * INTD], dn,
                                  preferred_element_type=jnp.float32
                                  ).astype(jnp.bfloat16)
    ott_ref[...] = (lax.dot_general(tt_ref[...], w[:, 3 * INTD:4 * INTD], dn,
                                    preferred_element_type=jnp.float32)
                    + b_ref[...]).astype(jnp.bfloat16)


def _project(test_emb, question_emb, tag_emb, testTag_emb, cat_W, cat_b):
    shapes = tuple(
        jax.ShapeDtypeStruct((t.shape[0], HALF), jnp.bfloat16)
        for t in (test_emb, question_emb, tag_emb, testTag_emb))
    return pl.pallas_call(
        _project_body,
        out_shape=shapes,
    )(test_emb, question_emb, tag_emb, testTag_emb, cat_W,
      cat_b.reshape(1, HALF))


NHALFC = NCHUNK // 2


NUMC = 2
NTOK = TPW // NUMC


def _gather_sum_body(it_hbm, iq_hbm, ig_hbm, itt_hbm, af_hbm, z_hbm,
                     tt_hbm, tq_hbm, tg_hbm, ttt_hbm,
                     out_hbm, num8_hbm,
                     iv_t, iv_q, iv_g, iv_tt,
                     sp_t, sp_q, sp_g, sp_tt,
                     a0, a1, a2, a3, oa,
                     b0, b1, b2, b3, ob,
                     av, r0, r1, r2, r3,
                     sga, sgb, soa, sob, sza, szb):
    wid = lax.axis_index("s") * NC + lax.axis_index("c")
    base = wid * TPW
    @pl.when(lax.axis_index("s") == 0)
    def _():
        pltpu.sync_copy(tt_hbm, sp_t)
        pltpu.sync_copy(tq_hbm, sp_q)
        pltpu.sync_copy(tg_hbm, sp_g)
        pltpu.sync_copy(ttt_hbm, sp_tt)

    pltpu.sync_copy(it_hbm.at[wid], iv_t)
    pltpu.sync_copy(iq_hbm.at[wid], iv_q)
    pltpu.sync_copy(ig_hbm.at[wid], iv_g)
    pltpu.sync_copy(itt_hbm.at[wid], iv_tt)
    plsc.subcore_barrier()

    ivs = (iv_t, iv_q, iv_g, iv_tt)
    tabs = (sp_t, sp_q, sp_g, sp_tt)

    def fire_zero(buf, sem):
        pltpu.async_copy(z_hbm, buf, sem)

    def wait_zero(buf, sem):
        pltpu.make_async_copy(z_hbm, buf, sem).wait()

    def fire_g(ci, bufs, sem):
        buf = bufs[0]
        for tab, iv in zip(tabs, ivs):
            pltpu.async_copy(tab.at[iv.at[ci]], buf, sem, add=True)

    def drain_g(ci, bufs, sem):
        buf = bufs[0]
        for tab, iv in zip(tabs, ivs):
            pltpu.make_async_copy(tab.at[iv.at[ci]], buf, sem).wait()

    def do_sum(bufs, o):
        u0 = bufs[0]

        def tok_body(i, carry2):
            for t in range(2):
                k = i + t * (CHUNK // 2)
                for j in range(HALF // 32):
                    s = pl.ds(j * 32, 32)
                    acc = u0[k, s]
                    w = plsc.bitcast(acc, jnp.int32)
                    lo = plsc.bitcast(w << 16, jnp.float32)
                    hi = plsc.bitcast(w & jnp.int32(-65536), jnp.float32)
                    o[i, pl.ds(t * HALF + j * 32, 16)] = lo
                    o[i, pl.ds(t * HALF + j * 32 + 16, 16)] = hi
            return carry2

        lax.fori_loop(0, CHUNK // 2, tok_body, 0)

    row0 = wid * (TPW // 2)
    CH2 = CHUNK // 2

    def fire_out(ci, o, sem):
        pltpu.async_copy(o, out_hbm.at[pl.ds(row0 + ci * CH2, CH2)], sem)

    def wait_out(o, sem):
        pltpu.make_async_copy(o, out_hbm.at[pl.ds(row0, CH2)], sem).wait()

    abufs = (a0, a1, a2, a3)
    bbufs = (b0, b1, b2, b3)

    pltpu.sync_copy(z_hbm, a0)
    fire_g(0, abufs, sga)
    pltpu.sync_copy(z_hbm, b0)
    fire_g(1, bbufs, sgb)

    def body(g, carry):
        c0 = 2 * g
        c1 = 2 * g + 1
        drain_g(c0, abufs, sga)

        @pl.when(g > 0)
        def _():
            wait_out(oa, soa)

        do_sum(abufs, oa)
        fire_out(c0, oa, soa)
        fire_zero(a0, sza)

        @pl.when(g < NHALFC - 1)
        def _():
            wait_zero(a0, sza)
            fire_g(c0 + 2, abufs, sga)

        drain_g(c1, bbufs, sgb)

        @pl.when(g > 0)
        def _():
            wait_out(ob, sob)

        do_sum(bbufs, ob)
        fire_out(c1, ob, sob)
        fire_zero(b0, szb)

        @pl.when(g < NHALFC - 1)
        def _():
            wait_zero(b0, szb)
            fire_g(c1 + 2, bbufs, sgb)

        return carry

    lax.fori_loop(0, NHALFC, body, 0)
    wait_zero(a0, sza)
    wait_zero(b0, szb)
    wait_out(oa, soa)
    wait_out(ob, sob)

    rbufs = (r0, r1, r2, r3)
    lanes = lax.iota(jnp.int32, 16)

    def num_chunk(nc, carry):
        pltpu.sync_copy(
            af_hbm.at[pl.ds(nc * 4 * (T // 2) + wid * 4 * NTOK, 4 * NTOK)], av)

        def vec_body(v, carry2):
            win = av.at[pl.ds(v * 64, 64)]
            for c in range(4):
                idx = lanes * 4 + c
                rbufs[c][pl.ds(v * 16, 16)] = plsc.load_gather(win, [idx])
            return carry2

        lax.fori_loop(0, NTOK // 16, vec_body, 0)
        for c in range(4):
            pltpu.sync_copy(
                rbufs[c],
                num8_hbm.at[4 * nc + c, pl.ds(wid * NTOK, NTOK)])
        return carry

    lax.fori_loop(0, NUMC, num_chunk, 0)


@functools.partial(jax.jit, static_argnums=())
def _gather_sum(it, iq, ig, itt, af, tab_t, tab_q, tab_g, tab_tt):
    mesh = plsc.VectorSubcoreMesh(core_axis_name="c", subcore_axis_name="s")
    row = pltpu.VMEM((CHUNK, HALF), jnp.bfloat16)
    orow = pltpu.VMEM((CHUNK // 2, HID), jnp.float32)
    f = pl.kernel(
        _gather_sum_body,
        out_type=(jax.ShapeDtypeStruct((T // 2, HID), jnp.float32),
                  jax.ShapeDtypeStruct((8, T // 2), jnp.float32)),
        mesh=mesh,
        compiler_params=pltpu.CompilerParams(use_tc_tiling_on_sc=False,
                                             needs_layout_passes=False),
        scratch_types=[
            pltpu.VMEM((NCHUNK, CHUNK), jnp.int32),
            pltpu.VMEM((NCHUNK, CHUNK), jnp.int32),
            pltpu.VMEM((NCHUNK, CHUNK), jnp.int32),
            pltpu.VMEM((NCHUNK, CHUNK), jnp.int32),
            pltpu.VMEM_SHARED(tab_t.shape, jnp.bfloat16),
            pltpu.VMEM_SHARED(tab_q.shape, jnp.bfloat16),
            pltpu.VMEM_SHARED(tab_g.shape, jnp.bfloat16),
            pltpu.VMEM_SHARED(tab_tt.shape, jnp.bfloat16),
            row, row, row, row, orow,
            row, row, row, row, orow,
            pltpu.VMEM((4 * NTOK,), jnp.float32),
            pltpu.VMEM((NTOK,), jnp.float32),
            pltpu.VMEM((NTOK,), jnp.float32),
            pltpu.VMEM((NTOK,), jnp.float32),
            pltpu.VMEM((NTOK,), jnp.float32),
            pltpu.SemaphoreType.DMA,
            pltpu.SemaphoreType.DMA,
            pltpu.SemaphoreType.DMA,
            pltpu.SemaphoreType.DMA,
            pltpu.SemaphoreType.DMA,
            pltpu.SemaphoreType.DMA,
        ],
    )
    zbuf = jnp.zeros((CHUNK, HALF), jnp.bfloat16)
    return f(it, iq, ig, itt, af, zbuf, tab_t, tab_q, tab_g, tab_tt)


TB = 4096
T2 = T // 2


def _dense_body(cat_ref, num_ref, w2_ref, m1_ref, pc_ref, pn_ref,
                eg1_ref, eg2_ref, cb2_ref, nb2_ref, nlnb2_ref,
                ob2_ref, out_ref):
    f32 = jnp.float32
    dn = (((1,), (0,)), ((), ()))

    def mm(a, b):
        return lax.dot_general(a, b, dn, preferred_element_type=f32)

    def ln(x, mask, eg, bias, inv_n):
        s = mm(x, mask) * inv_n
        q = mm(x * x, mask) * inv_n
        r = lax.rsqrt(q - s * s + LN_EPS)
        return x * mm(r, eg) + (mm(-s * r, eg) + bias)

    m1 = m1_ref[...]
    cat = cat_ref[...]
    catn = ln(cat, m1, eg1_ref[...], cb2_ref[...], 1.0 / HALF)

    dn0 = (((0,), (0,)), ((), ()))
    numv = lax.dot_general(num_ref[...], w2_ref[...], dn0,
                           preferred_element_type=f32) + nb2_ref[...]
    numn = ln(numv, m1, eg2_ref[...], nlnb2_ref[...], 1.0 / HALF)

    s3 = (mm(catn, m1) + mm(numn, m1)) * (1.0 / HID)
    q3 = (mm(catn * catn, m1) + mm(numn * numn, m1)) * (1.0 / HID)
    r3 = lax.rsqrt(q3 - s3 * s3 + LN_EPS)
    msr = -s3 * r3
    eg3c = pc_ref[...]
    eg3n = pn_ref[...]
    out_cat = catn * mm(r3, eg3c) + (mm(msr, eg3c) + ob2_ref[0:1, 0:HID])
    out_num = numn * mm(r3, eg3n) + (mm(msr, eg3n) + ob2_ref[0:1, HID:2 * HID])
    out_ref[0, :, 0:HALF] = out_cat[:, 0:HALF]
    out_ref[0, :, HALF:HID] = out_num[:, 0:HALF]
    out_ref[1, :, 0:HALF] = out_cat[:, HALF:HID]
    out_ref[1, :, HALF:HID] = out_num[:, HALF:HID]


def _dense(cat_pre, num8, num_W, num_b,
           cat_ln_g, cat_ln_b, num_ln_g, num_ln_b, out_ln_g, out_ln_b):
    f32 = jnp.float32
    two = lambda a: jnp.concatenate([a, a]).reshape(1, -1)
    w2 = jnp.kron(jnp.eye(2, dtype=f32), num_W.T)
    m1 = jnp.kron(jnp.eye(2, dtype=f32), jnp.ones((HALF, 1), f32))
    eg1 = m1.T * two(cat_ln_g)
    eg2 = m1.T * two(num_ln_g)
    eg3c = m1.T * two(out_ln_g[:HALF])
    eg3n = m1.T * two(out_ln_g[HALF:])
    ob2 = jnp.concatenate(
        [two(out_ln_b[:HALF]), two(out_ln_b[HALF:])], axis=1)
    vec = lambda n: pl.BlockSpec((1, n), lambda i: (0, 0))
    nb = T2 // TB
    return pl.pallas_call(
        _dense_body,
        grid=(nb,),
        in_specs=[
            pl.BlockSpec((TB, HID), lambda i: (i, 0)),
            pl.BlockSpec((8, TB), lambda i: (0, i)),
            pl.BlockSpec((8, HID), lambda i: (0, 0)),
            pl.BlockSpec((HID, 2), lambda i: (0, 0)),
            pl.BlockSpec((2, HID), lambda i: (0, 0)),
            pl.BlockSpec((2, HID), lambda i: (0, 0)),
            pl.BlockSpec((2, HID), lambda i: (0, 0)),
            pl.BlockSpec((2, HID), lambda i: (0, 0)),
            vec(HID), vec(HID), vec(HID),
            vec(2 * HID),
        ],
        out_specs=pl.BlockSpec((2, TB, HID), lambda i: (0, i, 0)),
        out_shape=jax.ShapeDtypeStruct((2, T2, HID), jnp.float32),
    )(cat_pre, num8, w2, m1, eg3c, eg3n, eg1, eg2,
      two(cat_ln_b), two(num_b), two(num_ln_b), ob2)


def kernel(current_test, current_question, current_tag, current_testTag,
           num_0, num_1, num_2, num_3,
           test_emb, question_emb, tag_emb, testTag_emb,
           cat_W, cat_b, cat_ln_g, cat_ln_b,
           num_W, num_b, num_ln_g, num_ln_b,
           out_ln_g, out_ln_b):
    lam = []
    for k in range(16):
        lam += [k, 16 + k]
    perm = jnp.asarray([g0 + x for g0 in (0, 32) for x in lam], jnp.int32)
    tab_t, tab_q, tab_g, tab_tt = _project(
        test_emb, question_emb, tag_emb, testTag_emb, cat_W[perm], cat_b[perm])

    def widx(a):
        return (a.reshape(2, NW, NCHUNK, CHUNK // 2)
                .transpose(1, 2, 0, 3).reshape(NW, NCHUNK, CHUNK))

    af = jnp.concatenate([num_0.reshape(-1), num_1.reshape(-1),
                          num_2.reshape(-1), num_3.reshape(-1)])

    cat_pre, num8 = _gather_sum(
        widx(current_test), widx(current_question),
        widx(current_tag), widx(current_testTag), af,
        tab_t, tab_q, tab_g, tab_tt)

    out = _dense(cat_pre, num8, num_W, num_b,
                 cat_ln_g, cat_ln_b, num_ln_g, num_ln_b, out_ln_g, out_ln_b)
    return out.reshape(B, L, HID)


# speedup vs baseline: 1.0036x; 1.0036x over previous
import functools

import jax
import jax.numpy as jnp
from jax import lax
from jax.experimental import pallas as pl
from jax.experimental.pallas import tpu as pltpu
from jax.experimental.pallas import tpu_sc as plsc

B, L = 1024, 200
T = B * L
HID = 128
INTD = 42
HALF = 64

NC, NS = 2, 16
NW = NC * NS
TPW = T // NW
CHUNK = 128
NCHUNK = TPW // CHUNK

LN_EPS = 1e-6


def _project_body(test_ref, q_ref, tag_ref, tt_ref, w_ref, b_ref,
                  ot_ref, oq_ref, og_ref, ott_ref):
    w = w_ref[...]
    dn = (((1,), (1,)), ((), ()))
    ot_ref[...] = lax.dot_general(test_ref[...], w[:, 0 * INTD:1 * INTD], dn,
                                  preferred_element_type=jnp.float32
                                  ).astype(jnp.bfloat16)
    oq_ref[...] = lax.dot_general(q_ref[...], w[:, 1 * INTD:2 * INTD], dn,
                                  preferred_element_type=jnp.float32
                                  ).astype(jnp.bfloat16)
    og_ref[...] = lax.dot_general(tag_ref[...], w[:, 2 * INTD:3 * INTD], dn,
                                  preferred_element_type=jnp.float32
                                  ).astype(jnp.bfloat16)
    ott_ref[...] = (lax.dot_general(tt_ref[...], w[:, 3 * INTD:4 * INTD], dn,
                                    preferred_element_type=jnp.float32)
                    + b_ref[...]).astype(jnp.bfloat16)


def _project(test_emb, question_emb, tag_emb, testTag_emb, cat_W, cat_b):
    shapes = tuple(
        jax.ShapeDtypeStruct((t.shape[0], HALF), jnp.bfloat16)
        for t in (test_emb, question_emb, tag_emb, testTag_emb))
    return pl.pallas_call(
        _project_body,
        out_shape=shapes,
    )(test_emb, question_emb, tag_emb, testTag_emb, cat_W,
      cat_b.reshape(1, HALF))


NHALFC = NCHUNK // 2


NUMC = 2
NTOK = TPW // NUMC


def _gather_sum_body(it_hbm, iq_hbm, ig_hbm, itt_hbm, af_hbm, z_hbm,
                     tt_hbm, tq_hbm, tg_hbm, ttt_hbm,
                     out_hbm, num8_hbm,
                     iv_t, iv_q, iv_g, iv_tt,
                     sp_t, sp_q, sp_g, sp_tt,
                     a0, a1, a2, a3, oa,
                     b0, b1, b2, b3, ob,
                     av, r0, r1, r2, r3,
                     sga, sgb, soa, sob, sza, szb):
    wid = lax.axis_index("s") * NC + lax.axis_index("c")
    base = wid * TPW
    @pl.when(lax.axis_index("s") == 0)
    def _():
        pltpu.sync_copy(tt_hbm, sp_t)
        pltpu.sync_copy(tq_hbm, sp_q)
        pltpu.sync_copy(tg_hbm, sp_g)
        pltpu.sync_copy(ttt_hbm, sp_tt)

    pltpu.sync_copy(it_hbm.at[wid], iv_t)
    pltpu.sync_copy(iq_hbm.at[wid], iv_q)
    pltpu.sync_copy(ig_hbm.at[wid], iv_g)
    pltpu.sync_copy(itt_hbm.at[wid], iv_tt)
    plsc.subcore_barrier()

    ivs = (iv_t, iv_q, iv_g, iv_tt)
    tabs = (sp_t, sp_q, sp_g, sp_tt)

    def fire_zero(buf, sem):
        pltpu.async_copy(z_hbm, buf, sem)

    def wait_zero(buf, sem):
        pltpu.make_async_copy(z_hbm, buf, sem).wait()

    def fire_g(ci, bufs, sem):
        buf = bufs[0]
        for tab, iv in zip(tabs, ivs):
            pltpu.async_copy(tab.at[iv.at[ci]], buf, sem, add=True)

    def drain_g(ci, bufs, sem):
        buf = bufs[0]
        for tab, iv in zip(tabs, ivs):
            pltpu.make_async_copy(tab.at[iv.at[ci]], buf, sem).wait()

    def do_sum(bufs, o):
        u0 = bufs[0]

        def tok_body(i, carry2):
            for t in range(2):
                k = i + t * (CHUNK // 2)
                for j in range(HALF // 32):
                    s = pl.ds(j * 32, 32)
                    acc = u0[k, s]
                    w = plsc.bitcast(acc, jnp.int32)
                    lo = plsc.bitcast(w << 16, jnp.float32)
                    hi = plsc.bitcast(w & jnp.int32(-65536), jnp.float32)
                    o[i, pl.ds(t * HALF + j * 32, 16)] = lo
                    o[i, pl.ds(t * HALF + j * 32 + 16, 16)] = hi
            return carry2

        lax.fori_loop(0, CHUNK // 2, tok_body, 0)

    row0 = wid * (TPW // 2)
    CH2 = CHUNK // 2

    def fire_out(ci, o, sem):
        pltpu.async_copy(o, out_hbm.at[pl.ds(row0 + ci * CH2, CH2)], sem)

    def wait_out(o, sem):
        pltpu.make_async_copy(o, out_hbm.at[pl.ds(row0, CH2)], sem).wait()

    abufs = (a0, a1, a2, a3)
    bbufs = (b0, b1, b2, b3)

    pltpu.sync_copy(z_hbm, a0)
    fire_g(0, abufs, sga)
    pltpu.sync_copy(z_hbm, b0)
    fire_g(1, bbufs, sgb)

    def body(g, carry):
        c0 = 2 * g
        c1 = 2 * g + 1
        drain_g(c0, abufs, sga)

        @pl.when(g > 0)
        def _():
            wait_out(oa, soa)

        do_sum(abufs, oa)
        fire_out(c0, oa, soa)
        fire_zero(a0, sza)

        @pl.when(g < NHALFC - 1)
        def _():
            wait_zero(a0, sza)
            fire_g(c0 + 2, abufs, sga)

        drain_g(c1, bbufs, sgb)

        @pl.when(g > 0)
        def _():
            wait_out(ob, sob)

        do_sum(bbufs, ob)
        fire_out(c1, ob, sob)
        fire_zero(b0, szb)

        @pl.when(g < NHALFC - 1)
        def _():
            wait_zero(b0, szb)
            fire_g(c1 + 2, bbufs, sgb)

        return carry

    lax.fori_loop(0, NHALFC, body, 0)
    wait_zero(a0, sza)
    wait_zero(b0, szb)
    wait_out(oa, soa)
    wait_out(ob, sob)

    rbufs = (r0, r1, r2, r3)
    lanes = lax.iota(jnp.int32, 16)

    def num_chunk(nc, carry):
        pltpu.sync_copy(
            af_hbm.at[pl.ds(nc * 4 * (T // 2) + wid * 4 * NTOK, 4 * NTOK)], av)

        def vec_body(v, carry2):
            win = av.at[pl.ds(v * 64, 64)]
            for c in range(4):
                idx = lanes * 4 + c
                rbufs[c][pl.ds(v * 16, 16)] = plsc.load_gather(win, [idx])
            return carry2

        lax.fori_loop(0, NTOK // 16, vec_body, 0)
        for c in range(4):
            pltpu.sync_copy(
                rbufs[c],
                num8_hbm.at[4 * nc + c, pl.ds(wid * NTOK, NTOK)])
        return carry

    lax.fori_loop(0, NUMC, num_chunk, 0)


@functools.partial(jax.jit, static_argnums=())
def _gather_sum(it, iq, ig, itt, af, tab_t, tab_q, tab_g, tab_tt):
    mesh = plsc.VectorSubcoreMesh(core_axis_name="c", subcore_axis_name="s")
    row = pltpu.VMEM((CHUNK, HALF), jnp.bfloat16)
    orow = pltpu.VMEM((CHUNK // 2, HID), jnp.float32)
    f = pl.kernel(
        _gather_sum_body,
        out_type=(jax.ShapeDtypeStruct((T // 2, HID), jnp.float32),
                  jax.ShapeDtypeStruct((8, T // 2), jnp.float32)),
        mesh=mesh,
        compiler_params=pltpu.CompilerParams(use_tc_tiling_on_sc=False,
                                             needs_layout_passes=False),
        scratch_types=[
            pltpu.VMEM((NCHUNK, CHUNK), jnp.int32),
            pltpu.VMEM((NCHUNK, CHUNK), jnp.int32),
            pltpu.VMEM((NCHUNK, CHUNK), jnp.int32),
            pltpu.VMEM((NCHUNK, CHUNK), jnp.int32),
            pltpu.VMEM_SHARED(tab_t.shape, jnp.bfloat16),
            pltpu.VMEM_SHARED(tab_q.shape, jnp.bfloat16),
            pltpu.VMEM_SHARED(tab_g.shape, jnp.bfloat16),
            pltpu.VMEM_SHARED(tab_tt.shape, jnp.bfloat16),
            row, row, row, row, orow,
            row, row, row, row, orow,
            pltpu.VMEM((4 * NTOK,), jnp.float32),
            pltpu.VMEM((NTOK,), jnp.float32),
            pltpu.VMEM((NTOK,), jnp.float32),
            pltpu.VMEM((NTOK,), jnp.float32),
            pltpu.VMEM((NTOK,), jnp.float32),
            pltpu.SemaphoreType.DMA,
            pltpu.SemaphoreType.DMA,
            pltpu.SemaphoreType.DMA,
            pltpu.SemaphoreType.DMA,
            pltpu.SemaphoreType.DMA,
            pltpu.SemaphoreType.DMA,
        ],
    )
    zbuf = jnp.zeros((CHUNK, HALF), jnp.bfloat16)
    return f(it, iq, ig, itt, af, zbuf, tab_t, tab_q, tab_g, tab_tt)


TB = 2048
T2 = T // 2


def _dense_body(cat_ref, num_ref, w2_ref, m1_ref, pc_ref, pn_ref,
                eg1_ref, eg2_ref, cb2_ref, nb2_ref, nlnb2_ref,
                ob2_ref, out_ref):
    f32 = jnp.float32
    dn = (((1,), (0,)), ((), ()))

    def mm(a, b):
        return lax.dot_general(a, b, dn, preferred_element_type=f32)

    def ln(x, mask, eg, bias, inv_n):
        s = mm(x, mask) * inv_n
        q = mm(x * x, mask) * inv_n
        r = lax.rsqrt(q - s * s + LN_EPS)
        return x * mm(r, eg) + (mm(-s * r, eg) + bias)

    m1 = m1_ref[...]
    cat = cat_ref[...]
    catn = ln(cat, m1, eg1_ref[...], cb2_ref[...], 1.0 / HALF)

    dn0 = (((0,), (0,)), ((), ()))
    numv = lax.dot_general(num_ref[...], w2_ref[...], dn0,
                           preferred_element_type=f32) + nb2_ref[...]
    numn = ln(numv, m1, eg2_ref[...], nlnb2_ref[...], 1.0 / HALF)

    s3 = (mm(catn, m1) + mm(numn, m1)) * (1.0 / HID)
    q3 = (mm(catn * catn, m1) + mm(numn * numn, m1)) * (1.0 / HID)
    r3 = lax.rsqrt(q3 - s3 * s3 + LN_EPS)
    msr = -s3 * r3
    eg3c = pc_ref[...]
    eg3n = pn_ref[...]
    out_cat = catn * mm(r3, eg3c) + (mm(msr, eg3c) + ob2_ref[0:1, 0:HID])
    out_num = numn * mm(r3, eg3n) + (mm(msr, eg3n) + ob2_ref[0:1, HID:2 * HID])
    out_ref[0, :, 0:HALF] = out_cat[:, 0:HALF]
    out_ref[0, :, HALF:HID] = out_num[:, 0:HALF]
    out_ref[1, :, 0:HALF] = out_cat[:, HALF:HID]
    out_ref[1, :, HALF:HID] = out_num[:, HALF:HID]


def _dense(cat_pre, num8, num_W, num_b,
           cat_ln_g, cat_ln_b, num_ln_g, num_ln_b, out_ln_g, out_ln_b):
    f32 = jnp.float32
    two = lambda a: jnp.concatenate([a, a]).reshape(1, -1)
    w2 = jnp.kron(jnp.eye(2, dtype=f32), num_W.T)
    m1 = jnp.kron(jnp.eye(2, dtype=f32), jnp.ones((HALF, 1), f32))
    eg1 = m1.T * two(cat_ln_g)
    eg2 = m1.T * two(num_ln_g)
    eg3c = m1.T * two(out_ln_g[:HALF])
    eg3n = m1.T * two(out_ln_g[HALF:])
    ob2 = jnp.concatenate(
        [two(out_ln_b[:HALF]), two(out_ln_b[HALF:])], axis=1)
    vec = lambda n: pl.BlockSpec((1, n), lambda i: (0, 0))
    nb = T2 // TB
    return pl.pallas_call(
        _dense_body,
        grid=(nb,),
        in_specs=[
            pl.BlockSpec((TB, HID), lambda i: (i, 0)),
            pl.BlockSpec((8, TB), lambda i: (0, i)),
            pl.BlockSpec((8, HID), lambda i: (0, 0)),
            pl.BlockSpec((HID, 2), lambda i: (0, 0)),
            pl.BlockSpec((2, HID), lambda i: (0, 0)),
            pl.BlockSpec((2, HID), lambda i: (0, 0)),
            pl.BlockSpec((2, HID), lambda i: (0, 0)),
            pl.BlockSpec((2, HID), lambda i: (0, 0)),
            vec(HID), vec(HID), vec(HID),
            vec(2 * HID),
        ],
        out_specs=pl.BlockSpec((2, TB, HID), lambda i: (0, i, 0)),
        out_shape=jax.ShapeDtypeStruct((2, T2, HID), jnp.float32),
    )(cat_pre, num8, w2, m1, eg3c, eg3n, eg1, eg2,
      two(cat_ln_b), two(num_b), two(num_ln_b), ob2)


def kernel(current_test, current_question, current_tag, current_testTag,
           num_0, num_1, num_2, num_3,
           test_emb, question_emb, tag_emb, testTag_emb,
           cat_W, cat_b, cat_ln_g, cat_ln_b,
           num_W, num_b, num_ln_g, num_ln_b,
           out_ln_g, out_ln_b):
    lam = []
    for k in range(16):
        lam += [k, 16 + k]
    perm = jnp.asarray([g0 + x for g0 in (0, 32) for x in lam], jnp.int32)
    tab_t, tab_q, tab_g, tab_tt = _project(
        test_emb, question_emb, tag_emb, testTag_emb, cat_W[perm], cat_b[perm])

    def widx(a):
        return (a.reshape(2, NW, NCHUNK, CHUNK // 2)
                .transpose(1, 2, 0, 3).reshape(NW, NCHUNK, CHUNK))

    af = jnp.concatenate([num_0.reshape(-1), num_1.reshape(-1),
                          num_2.reshape(-1), num_3.reshape(-1)])

    cat_pre, num8 = _gather_sum(
        widx(current_test), widx(current_question),
        widx(current_tag), widx(current_testTag), af,
        tab_t, tab_q, tab_g, tab_tt)

    out = _dense(cat_pre, num8, num_W, num_b,
                 cat_ln_g, cat_ln_b, num_ln_g, num_ln_b, out_ln_g, out_ln_b)
    return out.reshape(B, L, HID)


# speedup vs baseline: 1.2586x; 1.2541x over previous
import functools

import jax
import jax.numpy as jnp
from jax import lax
from jax.experimental import pallas as pl
from jax.experimental.pallas import tpu as pltpu
from jax.experimental.pallas import tpu_sc as plsc

B, L = 1024, 200
T = B * L
HID = 128
INTD = 42
HALF = 64

NC, NS = 2, 16
NW = NC * NS
TPW = T // NW
CHUNK = 128
NCHUNK = TPW // CHUNK

LN_EPS = 1e-6


def _project_body(test_ref, q_ref, tag_ref, tt_ref, w_ref, b_ref,
                  ot_ref, oq_ref, og_ref, ott_ref):
    w = w_ref[...]
    dn = (((1,), (1,)), ((), ()))
    ot_ref[...] = lax.dot_general(test_ref[...], w[:, 0 * INTD:1 * INTD], dn,
                                  preferred_element_type=jnp.float32
                                  ).astype(jnp.bfloat16)
    oq_ref[...] = lax.dot_general(q_ref[...], w[:, 1 * INTD:2 * INTD], dn,
                                  preferred_element_type=jnp.float32
                                  ).astype(jnp.bfloat16)
    og_ref[...] = lax.dot_general(tag_ref[...], w[:, 2 * INTD:3 * INTD], dn,
                                  preferred_element_type=jnp.float32
                                  ).astype(jnp.bfloat16)
    ott_ref[...] = (lax.dot_general(tt_ref[...], w[:, 3 * INTD:4 * INTD], dn,
                                    preferred_element_type=jnp.float32)
                    + b_ref[...]).astype(jnp.bfloat16)


def _project(test_emb, question_emb, tag_emb, testTag_emb, cat_W, cat_b):
    shapes = tuple(
        jax.ShapeDtypeStruct((t.shape[0], HALF), jnp.bfloat16)
        for t in (test_emb, question_emb, tag_emb, testTag_emb))
    return pl.pallas_call(
        _project_body,
        out_shape=shapes,
    )(test_emb, question_emb, tag_emb, testTag_emb, cat_W,
      cat_b.reshape(1, HALF))


NHALFC = NCHUNK // 2


NUMC = 2
NTOK = TPW // NUMC


def _gather_sum_body(it_hbm, iq_hbm, ig_hbm, itt_hbm, af_hbm,
                     tt_hbm, tq_hbm, tg_hbm, ttt_hbm,
                     out_hbm, num8_hbm,
                     iv_t, iv_q, iv_g, iv_tt,
                     sp_t, sp_q, sp_g, sp_tt,
                     a0, a1, a2, a3, oa,
                     b0, b1, b2, b3, ob,
                     av, r0, r1, r2, r3,
                     sga, sgb, soa, sob):
    wid = lax.axis_index("s") * NC + lax.axis_index("c")
    base = wid * TPW
    @pl.when(lax.axis_index("s") == 0)
    def _():
        pltpu.sync_copy(tt_hbm, sp_t)
        pltpu.sync_copy(tq_hbm, sp_q)
        pltpu.sync_copy(tg_hbm, sp_g)
        pltpu.sync_copy(ttt_hbm, sp_tt)

    pltpu.sync_copy(it_hbm.at[wid], iv_t)
    pltpu.sync_copy(iq_hbm.at[wid], iv_q)
    pltpu.sync_copy(ig_hbm.at[wid], iv_g)
    pltpu.sync_copy(itt_hbm.at[wid], iv_tt)
    plsc.subcore_barrier()

    ivs = (iv_t, iv_q, iv_g, iv_tt)
    tabs = (sp_t, sp_q, sp_g, sp_tt)

    zv = jnp.zeros((32,), jnp.bfloat16)

    def zero_g(bufs):
        buf = bufs[0]

        def zbody(k, c):
            for u in range(4):
                buf[k * 4 + u, pl.ds(0, 32)] = zv
                buf[k * 4 + u, pl.ds(32, 32)] = zv
            return c

        lax.fori_loop(0, CHUNK // 4, zbody, 0)

    def fire_g(ci, bufs, sem):
        buf = bufs[0]
        for tab, iv in zip(tabs, ivs):
            pltpu.async_copy(tab.at[iv.at[ci]], buf, sem, add=True)

    def drain_g(ci, bufs, sem):
        buf = bufs[0]
        for tab, iv in zip(tabs, ivs):
            pltpu.make_async_copy(tab.at[iv.at[ci]], buf, sem).wait()

    def do_sum(bufs, o):
        u0 = bufs[0]

        def tok_body(i2, carry2):
            for v in range(2):
              i = i2 * 2 + v
              for t in range(2):
                k = i + t * (CHUNK // 2)
                for j in range(HALF // 32):
                    s = pl.ds(j * 32, 32)
                    acc = u0[k, s]
                    w = plsc.bitcast(acc, jnp.int32)
                    lo = plsc.bitcast(w << 16, jnp.float32)
                    hi = plsc.bitcast(w & jnp.int32(-65536), jnp.float32)
                    o[i, pl.ds(t * HALF + j * 32, 16)] = lo
                    o[i, pl.ds(t * HALF + j * 32 + 16, 16)] = hi
            return carry2

        lax.fori_loop(0, CHUNK // 4, tok_body, 0)

    row0 = wid * (TPW // 2)
    CH2 = CHUNK // 2

    def fire_out(ci, o, sem):
        pltpu.async_copy(o, out_hbm.at[pl.ds(row0 + ci * CH2, CH2)], sem)

    def wait_out(o, sem):
        pltpu.make_async_copy(o, out_hbm.at[pl.ds(row0, CH2)], sem).wait()

    abufs = (a0, a1, a2, a3)
    bbufs = (b0, b1, b2, b3)

    zero_g(abufs)
    fire_g(0, abufs, sga)

    def body(g, carry):
        c0 = 2 * g
        c1 = 2 * g + 1
        zero_g(bbufs)
        fire_g(c1, bbufs, sgb)
        drain_g(c0, abufs, sga)

        @pl.when(g > 0)
        def _():
            wait_out(oa, soa)

        do_sum(abufs, oa)
        fire_out(c0, oa, soa)

        @pl.when(g < NHALFC - 1)
        def _():
            zero_g(abufs)
            fire_g(c0 + 2, abufs, sga)

        drain_g(c1, bbufs, sgb)

        @pl.when(g > 0)
        def _():
            wait_out(ob, sob)

        do_sum(bbufs, ob)
        fire_out(c1, ob, sob)
        return carry

    lax.fori_loop(0, NHALFC, body, 0)
    wait_out(oa, soa)
    wait_out(ob, sob)

    rbufs = (r0, r1, r2, r3)
    lanes = lax.iota(jnp.int32, 16)

    def num_chunk(nc, carry):
        pltpu.sync_copy(
            af_hbm.at[pl.ds(nc * 4 * (T // 2) + wid * 4 * NTOK, 4 * NTOK)], av)

        def vec_body(v, carry2):
            win = av.at[pl.ds(v * 64, 64)]
            for c in range(4):
                idx = lanes * 4 + c
                rbufs[c][pl.ds(v * 16, 16)] = plsc.load_gather(win, [idx])
            return carry2

        lax.fori_loop(0, NTOK // 16, vec_body, 0)
        for c in range(4):
            pltpu.sync_copy(
                rbufs[c],
                num8_hbm.at[4 * nc + c, pl.ds(wid * NTOK, NTOK)])
        return carry

    lax.fori_loop(0, NUMC, num_chunk, 0)


@functools.partial(jax.jit, static_argnums=())
def _gather_sum(it, iq, ig, itt, af, tab_t, tab_q, tab_g, tab_tt):
    mesh = plsc.VectorSubcoreMesh(core_axis_name="c", subcore_axis_name="s")
    row = pltpu.VMEM((CHUNK, HALF), jnp.bfloat16)
    orow = pltpu.VMEM((CHUNK // 2, HID), jnp.float32)
    f = pl.kernel(
        _gather_sum_body,
        out_type=(jax.ShapeDtypeStruct((T // 2, HID), jnp.float32),
                  jax.ShapeDtypeStruct((8, T // 2), jnp.float32)),
        mesh=mesh,
        compiler_params=pltpu.CompilerParams(use_tc_tiling_on_sc=False,
                                             needs_layout_passes=False),
        scratch_types=[
            pltpu.VMEM((NCHUNK, CHUNK), jnp.int32),
            pltpu.VMEM((NCHUNK, CHUNK), jnp.int32),
            pltpu.VMEM((NCHUNK, CHUNK), jnp.int32),
            pltpu.VMEM((NCHUNK, CHUNK), jnp.int32),
            pltpu.VMEM_SHARED(tab_t.shape, jnp.bfloat16),
            pltpu.VMEM_SHARED(tab_q.shape, jnp.bfloat16),
            pltpu.VMEM_SHARED(tab_g.shape, jnp.bfloat16),
            pltpu.VMEM_SHARED(tab_tt.shape, jnp.bfloat16),
            row, row, row, row, orow,
            row, row, row, row, orow,
            pltpu.VMEM((4 * NTOK,), jnp.float32),
            pltpu.VMEM((NTOK,), jnp.float32),
            pltpu.VMEM((NTOK,), jnp.float32),
            pltpu.VMEM((NTOK,), jnp.float32),
            pltpu.VMEM((NTOK,), jnp.float32),
            pltpu.SemaphoreType.DMA,
            pltpu.SemaphoreType.DMA,
            pltpu.SemaphoreType.DMA,
            pltpu.SemaphoreType.DMA,
        ],
    )
    return f(it, iq, ig, itt, af, tab_t, tab_q, tab_g, tab_tt)


TB = 2048
T2 = T // 2


def _dense_body(cat_ref, num_ref, w2_ref, m1_ref, pc_ref, pn_ref,
                eg1_ref, eg2_ref, cb2_ref, nb2_ref, nlnb2_ref,
                ob2_ref, out_ref):
    f32 = jnp.float32
    dn = (((1,), (0,)), ((), ()))

    def mm(a, b):
        return lax.dot_general(a, b, dn, preferred_element_type=f32)

    def ln(x, mask, eg, bias, inv_n):
        s = mm(x, mask) * inv_n
        q = mm(x * x, mask) * inv_n
        r = lax.rsqrt(q - s * s + LN_EPS)
        return x * mm(r, eg) + (mm(-s * r, eg) + bias)

    m1 = m1_ref[...]
    cat = cat_ref[...]
    catn = ln(cat, m1, eg1_ref[...], cb2_ref[...], 1.0 / HALF)

    dn0 = (((0,), (0,)), ((), ()))
    numv = lax.dot_general(num_ref[...], w2_ref[...], dn0,
                           preferred_element_type=f32) + nb2_ref[...]
    numn = ln(numv, m1, eg2_ref[...], nlnb2_ref[...], 1.0 / HALF)

    s3 = (mm(catn, m1) + mm(numn, m1)) * (1.0 / HID)
    q3 = (mm(catn * catn, m1) + mm(numn * numn, m1)) * (1.0 / HID)
    r3 = lax.rsqrt(q3 - s3 * s3 + LN_EPS)
    msr = -s3 * r3
    eg3c = pc_ref[...]
    eg3n = pn_ref[...]
    out_cat = catn * mm(r3, eg3c) + (mm(msr, eg3c) + ob2_ref[0:1, 0:HID])
    out_num = numn * mm(r3, eg3n) + (mm(msr, eg3n) + ob2_ref[0:1, HID:2 * HID])
    out_ref[0, :, 0:HALF] = out_cat[:, 0:HALF]
    out_ref[0, :, HALF:HID] = out_num[:, 0:HALF]
    out_ref[1, :, 0:HALF] = out_cat[:, HALF:HID]
    out_ref[1, :, HALF:HID] = out_num[:, HALF:HID]


def _dense(cat_pre, num8, num_W, num_b,
           cat_ln_g, cat_ln_b, num_ln_g, num_ln_b, out_ln_g, out_ln_b):
    f32 = jnp.float32
    two = lambda a: jnp.concatenate([a, a]).reshape(1, -1)
    w2 = jnp.kron(jnp.eye(2, dtype=f32), num_W.T)
    m1 = jnp.kron(jnp.eye(2, dtype=f32), jnp.ones((HALF, 1), f32))
    eg1 = m1.T * two(cat_ln_g)
    eg2 = m1.T * two(num_ln_g)
    eg3c = m1.T * two(out_ln_g[:HALF])
    eg3n = m1.T * two(out_ln_g[HALF:])
    ob2 = jnp.concatenate(
        [two(out_ln_b[:HALF]), two(out_ln_b[HALF:])], axis=1)
    vec = lambda n: pl.BlockSpec((1, n), lambda i: (0, 0))
    nb = T2 // TB
    return pl.pallas_call(
        _dense_body,
        grid=(nb,),
        in_specs=[
            pl.BlockSpec((TB, HID), lambda i: (i, 0)),
            pl.BlockSpec((8, TB), lambda i: (0, i)),
            pl.BlockSpec((8, HID), lambda i: (0, 0)),
            pl.BlockSpec((HID, 2), lambda i: (0, 0)),
            pl.BlockSpec((2, HID), lambda i: (0, 0)),
            pl.BlockSpec((2, HID), lambda i: (0, 0)),
            pl.BlockSpec((2, HID), lambda i: (0, 0)),
            pl.BlockSpec((2, HID), lambda i: (0, 0)),
            vec(HID), vec(HID), vec(HID),
            vec(2 * HID),
        ],
        out_specs=pl.BlockSpec((2, TB, HID), lambda i: (0, i, 0)),
        out_shape=jax.ShapeDtypeStruct((2, T2, HID), jnp.float32),
    )(cat_pre, num8, w2, m1, eg3c, eg3n, eg1, eg2,
      two(cat_ln_b), two(num_b), two(num_ln_b), ob2)


def kernel(current_test, current_question, current_tag, current_testTag,
           num_0, num_1, num_2, num_3,
           test_emb, question_emb, tag_emb, testTag_emb,
           cat_W, cat_b, cat_ln_g, cat_ln_b,
           num_W, num_b, num_ln_g, num_ln_b,
           out_ln_g, out_ln_b):
    lam = []
    for k in range(16):
        lam += [k, 16 + k]
    perm = jnp.asarray([g0 + x for g0 in (0, 32) for x in lam], jnp.int32)
    tab_t, tab_q, tab_g, tab_tt = _project(
        test_emb, question_emb, tag_emb, testTag_emb, cat_W[perm], cat_b[perm])

    def widx(a):
        return (a.reshape(2, NW, NCHUNK, CHUNK // 2)
                .transpose(1, 2, 0, 3).reshape(NW, NCHUNK, CHUNK))

    af = jnp.concatenate([num_0.reshape(-1), num_1.reshape(-1),
                          num_2.reshape(-1), num_3.reshape(-1)])

    cat_pre, num8 = _gather_sum(
        widx(current_test), widx(current_question),
        widx(current_tag), widx(current_testTag), af,
        tab_t, tab_q, tab_g, tab_tt)

    out = _dense(cat_pre, num8, num_W, num_b,
                 cat_ln_g, cat_ln_b, num_ln_g, num_ln_b, out_ln_g, out_ln_b)
    return out.reshape(B, L, HID)
